# Initial kernel scaffold; baseline (speedup 1.0000x reference)
import jax
import jax.numpy as jnp
from jax.experimental import pallas as pl


def kernel(scores_raw, bbox_deltas, im_info):
    raise NotImplementedError("write your pallas kernel here")


# speedup vs baseline: 14.5766x; 14.5766x over previous
import functools

import jax
import jax.numpy as jnp
import numpy as np
from jax.experimental import pallas as pl
from jax.experimental.pallas import tpu as pltpu

_A = 9
_STRIDE = 16
_PRE = 6000
_POST = 300
_THRESH = 0.7
_NEG = -1e30


def _host_anchors(feature_h, feature_w):
    base_size = 16.0
    ratios = np.array([0.5, 1.0, 2.0])
    scales = np.array([8.0, 16.0, 32.0])
    x_ctr = (base_size - 1.0) / 2.0
    y_ctr = (base_size - 1.0) / 2.0
    size = base_size * base_size
    rows = []
    for r in ratios:
        ws = np.round(np.sqrt(size / r))
        hs = np.round(ws * r)
        for s in scales:
            w = ws * s
            h = hs * s
            rows.append([x_ctr - 0.5 * (w - 1.0), y_ctr - 0.5 * (h - 1.0),
                         x_ctr + 0.5 * (w - 1.0), y_ctr + 0.5 * (h - 1.0)])
    base = np.asarray(rows, dtype=np.float32)
    shift_x = np.arange(feature_w, dtype=np.float32) * _STRIDE
    shift_y = np.arange(feature_h, dtype=np.float32) * _STRIDE
    sx, sy = np.meshgrid(shift_x, shift_y)
    shifts = np.stack([sx.ravel(), sy.ravel(), sx.ravel(), sy.ravel()], axis=1).astype(np.float32)
    return (shifts[:, None, :] + base[None, :, :]).reshape(-1, 4)


def _body(sc_ref, dx_ref, dy_ref, dw_ref, dh_ref,
          ax1_ref, ay1_ref, ax2_ref, ay2_ref, h_ref, w_ref,
          out_ref,
          x1_s, y1_s, x2_s, y2_s, ar_s, msc_s, key_s):
    B, N = sc_ref.shape
    iota = jax.lax.broadcasted_iota(jnp.int32, (B, N), 1)

    ax1 = ax1_ref[...]
    ay1 = ay1_ref[...]
    ax2 = ax2_ref[...]
    ay2 = ay2_ref[...]
    aw = ax2 - ax1 + 1.0
    ah = ay2 - ay1 + 1.0
    acx = ax1 + 0.5 * aw
    acy = ay1 + 0.5 * ah
    pcx = dx_ref[...] * aw + acx
    pcy = dy_ref[...] * ah + acy
    pw = jnp.exp(dw_ref[...]) * aw
    ph = jnp.exp(dh_ref[...]) * ah
    px1 = pcx - 0.5 * pw
    py1 = pcy - 0.5 * ph
    px2 = pcx + 0.5 * pw
    py2 = pcy + 0.5 * ph
    hh = h_ref[...]
    ww = w_ref[...]
    x1 = jnp.clip(px1, 0.0, ww - 1.0)
    y1 = jnp.clip(py1, 0.0, hh - 1.0)
    x2 = jnp.clip(px2, 0.0, ww - 1.0)
    y2 = jnp.clip(py2, 0.0, hh - 1.0)
    x1_s[...] = x1
    y1_s[...] = y1
    x2_s[...] = x2
    y2_s[...] = y2
    ar_s[...] = (x2 - x1 + 1.0) * (y2 - y1 + 1.0)

    sc = sc_ref[...]
    bits = jax.lax.bitcast_convert_type(sc, jnp.int32)
    key = jnp.where(bits < 0, bits ^ jnp.int32(0x7FFFFFFF), bits)
    key_s[...] = key
    min32 = jnp.int32(-2147483648)

    def bs_val(i, u):
        b = 31 - i
        cand_u = u | jnp.left_shift(jnp.int32(1), b)
        thr = min32 + cand_u
        cnt = jnp.sum((key_s[...] >= thr).astype(jnp.int32), axis=1, keepdims=True)
        return jnp.where(cnt >= _PRE, cand_u, u)

    u = jax.lax.fori_loop(0, 32, bs_val, jnp.zeros((B, 1), jnp.int32))
    t6 = min32 + u

    keyv = key_s[...]
    cnt_gt = jnp.sum((keyv > t6).astype(jnp.int32), axis=1, keepdims=True)
    quota = _PRE - cnt_gt

    def bs_idx(i, lohi):
        lo, hi = lohi
        mid = (lo + hi) >> 1
        g = jnp.sum(((key_s[...] == t6) & (iota <= mid)).astype(jnp.int32),
                    axis=1, keepdims=True)
        ok = g >= quota
        return jnp.where(ok, lo, mid + 1), jnp.where(ok, mid, hi)

    lo0 = jnp.zeros((B, 1), jnp.int32)
    hi0 = jnp.full((B, 1), N - 1, jnp.int32)
    _, bound = jax.lax.fori_loop(0, 17, bs_idx, (lo0, hi0))

    sel = (keyv > t6) | ((keyv == t6) & (iota <= bound))
    msc_s[...] = jnp.where(sel, sc, jnp.float32(_NEG))

    bcol = jax.lax.broadcasted_iota(jnp.int32, (B, 1), 0).astype(jnp.float32)

    def nms(i, _):
        msc = msc_s[...]
        m = jnp.max(msc, axis=1, keepdims=True)
        eq = msc == m
        idx = jnp.min(jnp.where(eq, iota, jnp.int32(N)), axis=1, keepdims=True)
        selm = iota == idx
        x1 = x1_s[...]
        y1 = y1_s[...]
        x2 = x2_s[...]
        y2 = y2_s[...]
        fill = jnp.float32(-3.0e38)
        cx1 = jnp.max(jnp.where(selm, x1, fill), axis=1, keepdims=True)
        cy1 = jnp.max(jnp.where(selm, y1, fill), axis=1, keepdims=True)
        cx2 = jnp.max(jnp.where(selm, x2, fill), axis=1, keepdims=True)
        cy2 = jnp.max(jnp.where(selm, y2, fill), axis=1, keepdims=True)
        carea = (cx2 - cx1 + 1.0) * (cy2 - cy1 + 1.0)
        valid = (m > jnp.float32(_NEG * 0.5)).astype(jnp.float32)
        xx1 = jnp.maximum(cx1, x1)
        yy1 = jnp.maximum(cy1, y1)
        xx2 = jnp.minimum(cx2, x2)
        yy2 = jnp.minimum(cy2, y2)
        iw = jnp.maximum(xx2 - xx1 + 1.0, 0.0)
        ih = jnp.maximum(yy2 - yy1 + 1.0, 0.0)
        inter = iw * ih
        iou = inter / (carea + ar_s[...] - inter)
        msc_s[...] = jnp.where((iou > _THRESH) | selm, jnp.float32(_NEG), msc)
        row = jnp.concatenate(
            [bcol, cx1 * valid, cy1 * valid, cx2 * valid, cy2 * valid], axis=1)
        out_ref[i, :, :] = row
        return 0

    jax.lax.fori_loop(0, _POST, nms, 0)


@functools.partial(jax.jit, static_argnames=())
def kernel(scores_raw, bbox_deltas, im_info):
    B = scores_raw.shape[0]
    H, W = scores_raw.shape[2], scores_raw.shape[3]
    N = H * W * _A
    f32 = jnp.float32

    sc = jnp.transpose(scores_raw[:, _A:], (0, 2, 3, 1)).reshape(B, N)
    d = jnp.transpose(bbox_deltas, (0, 2, 3, 1)).reshape(B, N, 4)
    dx, dy, dw, dh = d[..., 0], d[..., 1], d[..., 2], d[..., 3]

    anch = _host_anchors(H, W)
    ax1 = jnp.asarray(anch[:, 0]).reshape(1, N)
    ay1 = jnp.asarray(anch[:, 1]).reshape(1, N)
    ax2 = jnp.asarray(anch[:, 2]).reshape(1, N)
    ay2 = jnp.asarray(anch[:, 3]).reshape(1, N)
    hcol = im_info[:, 0:1].astype(f32)
    wcol = im_info[:, 1:2].astype(f32)

    out = pl.pallas_call(
        _body,
        out_shape=jax.ShapeDtypeStruct((_POST, B, 5), f32),
        in_specs=[pl.BlockSpec(memory_space=pltpu.VMEM)] * 11,
        out_specs=pl.BlockSpec(memory_space=pltpu.VMEM),
        scratch_shapes=[
            pltpu.VMEM((B, N), f32),
            pltpu.VMEM((B, N), f32),
            pltpu.VMEM((B, N), f32),
            pltpu.VMEM((B, N), f32),
            pltpu.VMEM((B, N), f32),
            pltpu.VMEM((B, N), f32),
            pltpu.VMEM((B, N), jnp.int32),
        ],
    )(sc, dx, dy, dw, dh, ax1, ay1, ax2, ay2, hcol, wcol)
    return jnp.transpose(out, (1, 0, 2))
